# baseline (speedup 1.0000x reference)
import jax
import jax.numpy as jnp
from jax.experimental import pallas as pl
from jax.experimental.pallas import tpu as pltpu

N = 10
E_PAD = 96
W2_CHUNKS = 4


def _fused_kernel(ei_hbm, x_hbm, w1_hbm, b1_hbm, w2_hbm, b2_hbm, wct_hbm,
                  bc_hbm, out_ref,
                  ei_v, x_v, w1_v, b1_v, w2_v, b2_v, wct_v, bc_v,
                  s_ei, s_x, s_w1, s_b1, s_w2, s_b2, s_wct, s_bc):
    f32 = jnp.float32
    hid = w2_v.shape[0]
    ch = hid // W2_CHUNKS
    cp_ei = pltpu.make_async_copy(ei_hbm, ei_v, s_ei)
    cp_x = pltpu.make_async_copy(x_hbm, x_v, s_x)
    cp_w1 = pltpu.make_async_copy(w1_hbm, w1_v, s_w1)
    cp_b1 = pltpu.make_async_copy(b1_hbm, b1_v, s_b1)
    cp_w2 = [pltpu.make_async_copy(w2_hbm.at[pl.ds(i * ch, ch), :],
                                   w2_v.at[pl.ds(i * ch, ch), :],
                                   s_w2.at[i]) for i in range(W2_CHUNKS)]
    cp_b2 = pltpu.make_async_copy(b2_hbm, b2_v, s_b2)
    cp_wct = pltpu.make_async_copy(wct_hbm, wct_v, s_wct)
    cp_bc = pltpu.make_async_copy(bc_hbm, bc_v, s_bc)
    cp_ei.start()
    cp_x.start()
    cp_w1.start()
    cp_b1.start()
    for cp in cp_w2:
        cp.start()
    cp_b2.start()
    cp_wct.start()
    cp_bc.start()

    cp_ei.wait()
    src = ei_v[0:1, :]
    dst = ei_v[1:2, :]
    node_col = jax.lax.broadcasted_iota(jnp.int32, (N, E_PAD), 0)
    St = (src == node_col).astype(f32)
    Dt = (dst == node_col).astype(f32)

    deg = 1.0 + jnp.sum(Dt, axis=1, keepdims=True)
    dis = jax.lax.rsqrt(deg)
    dis_src = jnp.sum(St * dis, axis=0, keepdims=True)
    dis_dst = jnp.sum(Dt * dis, axis=0, keepdims=True)
    norm = dis_src * dis_dst

    A = jax.lax.dot_general(Dt * norm, St, (((1,), (1,)), ((), ())),
                            preferred_element_type=f32)
    eye = (jax.lax.broadcasted_iota(jnp.int32, (N, N), 0)
           == jax.lax.broadcasted_iota(jnp.int32, (N, N), 1)).astype(f32)
    A = A + eye * (1.0 / deg)

    cp_x.wait()
    cp_w1.wait()
    xw = jnp.dot(x_v[:, :], w1_v[:, :], preferred_element_type=f32)
    cp_b1.wait()
    h1 = jnp.maximum(jnp.dot(A, xw, preferred_element_type=f32)
                     + b1_v[:, :], 0.0)

    hw = jnp.zeros((N, hid), f32)
    for i in range(W2_CHUNKS):
        cp_w2[i].wait()
        hw = hw + jnp.dot(h1[:, i * ch:(i + 1) * ch],
                          w2_v[i * ch:(i + 1) * ch, :],
                          preferred_element_type=f32)
    cp_b2.wait()
    h2 = jnp.maximum(jnp.dot(A, hw, preferred_element_type=f32)
                     + b2_v[:, :], 0.0)

    cp_wct.wait()
    parts = [jnp.sum(h2 * wct_v[c]).reshape(1, 1)
             for c in range(wct_v.shape[0])]
    cp_bc.wait()
    logits = bc_v[:, :] + jnp.concatenate(parts, axis=1)

    m = jnp.max(logits, axis=1, keepdims=True)
    p = jnp.exp(logits - m)
    out_ref[:, :] = p / jnp.sum(p, axis=1, keepdims=True)


@jax.jit
def kernel(x, edge_index, W1, b1, W2, b2, Wc, bc):
    E = edge_index.shape[1]
    ei = edge_index.astype(jnp.int32)
    pad = jnp.full((2, E_PAD - E), -1, dtype=jnp.int32)
    ei = jnp.concatenate([ei, pad], axis=1)
    inf, hid = W1.shape
    ncls = Wc.shape[1]
    wct = jnp.transpose(Wc).reshape(ncls, N, hid)
    vmem = pl.BlockSpec(memory_space=pltpu.MemorySpace.VMEM)
    hbm = pl.BlockSpec(memory_space=pltpu.MemorySpace.HBM)
    out = pl.pallas_call(
        _fused_kernel,
        out_shape=jax.ShapeDtypeStruct((1, ncls), jnp.float32),
        in_specs=[hbm] * 8,
        out_specs=vmem,
        scratch_shapes=[
            pltpu.VMEM((2, E_PAD), jnp.int32),
            pltpu.VMEM((N, inf), jnp.float32),
            pltpu.VMEM((inf, hid), jnp.float32),
            pltpu.VMEM((1, hid), jnp.float32),
            pltpu.VMEM((hid, hid), jnp.float32),
            pltpu.VMEM((1, hid), jnp.float32),
            pltpu.VMEM((ncls, N, hid), jnp.float32),
            pltpu.VMEM((1, ncls), jnp.float32),
            pltpu.SemaphoreType.DMA,
            pltpu.SemaphoreType.DMA,
            pltpu.SemaphoreType.DMA,
            pltpu.SemaphoreType.DMA,
            pltpu.SemaphoreType.DMA((W2_CHUNKS,)),
            pltpu.SemaphoreType.DMA,
            pltpu.SemaphoreType.DMA,
            pltpu.SemaphoreType.DMA,
        ],
    )(ei, x, W1, b1.reshape(1, hid), W2, b2.reshape(1, hid), wct,
      bc.reshape(1, ncls))
    return out


# speedup vs baseline: 1.0308x; 1.0308x over previous
import jax
import jax.numpy as jnp
from jax.experimental import pallas as pl
from jax.experimental.pallas import tpu as pltpu

N = 10
E_PAD = 96


def _fused_kernel(ei_hbm, x_hbm, w1_hbm, b1_hbm, w2_hbm, b2_hbm, wct_hbm,
                  bc_hbm, out_ref,
                  ei_v, x_v, w1_v, b1_v, w2_v, b2_v, wct_v, bc_v,
                  s_ei, s_x, s_w1, s_b1, s_w2, s_b2, s_wct, s_bc):
    f32 = jnp.float32
    hid = w2_v.shape[0]
    cp_ei = pltpu.make_async_copy(ei_hbm, ei_v, s_ei)
    cp_x = pltpu.make_async_copy(x_hbm, x_v, s_x)
    cp_w1 = pltpu.make_async_copy(w1_hbm, w1_v, s_w1)
    cp_b1 = pltpu.make_async_copy(b1_hbm, b1_v, s_b1)
    cp_w2 = pltpu.make_async_copy(w2_hbm, w2_v, s_w2)
    cp_b2 = pltpu.make_async_copy(b2_hbm, b2_v, s_b2)
    cp_wct = pltpu.make_async_copy(wct_hbm, wct_v, s_wct)
    cp_bc = pltpu.make_async_copy(bc_hbm, bc_v, s_bc)
    cp_ei.start()
    cp_x.start()
    cp_w1.start()
    cp_b1.start()
    cp_w2.start()
    cp_b2.start()
    cp_wct.start()
    cp_bc.start()

    cp_ei.wait()
    src = ei_v[0:1, :]
    dst = ei_v[1:2, :]
    node_col = jax.lax.broadcasted_iota(jnp.int32, (N, E_PAD), 0)
    St = (src == node_col).astype(f32)
    Dt = (dst == node_col).astype(f32)

    deg = 1.0 + jnp.sum(Dt, axis=1, keepdims=True)
    dis = jax.lax.rsqrt(deg)
    dis_src = jnp.sum(St * dis, axis=0, keepdims=True)
    dis_dst = jnp.sum(Dt * dis, axis=0, keepdims=True)
    norm = dis_src * dis_dst

    A = jax.lax.dot_general(Dt * norm, St, (((1,), (1,)), ((), ())),
                            preferred_element_type=f32)
    eye = (jax.lax.broadcasted_iota(jnp.int32, (N, N), 0)
           == jax.lax.broadcasted_iota(jnp.int32, (N, N), 1)).astype(f32)
    A = A + eye * (1.0 / deg)

    cp_x.wait()
    cp_w1.wait()
    xw = jnp.dot(x_v[:, :], w1_v[:, :], preferred_element_type=f32)
    cp_b1.wait()
    h1 = jnp.maximum(jnp.dot(A, xw, preferred_element_type=f32)
                     + b1_v[:, :], 0.0)

    cp_w2.wait()
    hw = jnp.dot(h1, w2_v[:, :], preferred_element_type=f32)
    cp_b2.wait()
    h2 = jnp.maximum(jnp.dot(A, hw, preferred_element_type=f32)
                     + b2_v[:, :], 0.0)

    cp_wct.wait()
    parts = [jnp.sum(h2 * wct_v[c]).reshape(1, 1)
             for c in range(wct_v.shape[0])]
    cp_bc.wait()
    logits = bc_v[:, :] + jnp.concatenate(parts, axis=1)

    m = jnp.max(logits, axis=1, keepdims=True)
    p = jnp.exp(logits - m)
    out_ref[:, :] = p / jnp.sum(p, axis=1, keepdims=True)


@jax.jit
def kernel(x, edge_index, W1, b1, W2, b2, Wc, bc):
    E = edge_index.shape[1]
    ei = edge_index.astype(jnp.int32)
    pad = jnp.full((2, E_PAD - E), -1, dtype=jnp.int32)
    ei = jnp.concatenate([ei, pad], axis=1)
    inf, hid = W1.shape
    ncls = Wc.shape[1]
    wct = jnp.transpose(Wc).reshape(ncls, N, hid)
    vmem = pl.BlockSpec(memory_space=pltpu.MemorySpace.VMEM)
    hbm = pl.BlockSpec(memory_space=pltpu.MemorySpace.HBM)
    out = pl.pallas_call(
        _fused_kernel,
        out_shape=jax.ShapeDtypeStruct((1, ncls), jnp.float32),
        in_specs=[hbm] * 8,
        out_specs=vmem,
        scratch_shapes=[
            pltpu.VMEM((2, E_PAD), jnp.int32),
            pltpu.VMEM((N, inf), jnp.float32),
            pltpu.VMEM((inf, hid), jnp.float32),
            pltpu.VMEM((1, hid), jnp.float32),
            pltpu.VMEM((hid, hid), jnp.float32),
            pltpu.VMEM((1, hid), jnp.float32),
            pltpu.VMEM((ncls, N, hid), jnp.float32),
            pltpu.VMEM((1, ncls), jnp.float32),
            pltpu.SemaphoreType.DMA,
            pltpu.SemaphoreType.DMA,
            pltpu.SemaphoreType.DMA,
            pltpu.SemaphoreType.DMA,
            pltpu.SemaphoreType.DMA,
            pltpu.SemaphoreType.DMA,
            pltpu.SemaphoreType.DMA,
            pltpu.SemaphoreType.DMA,
        ],
    )(ei, x, W1, b1.reshape(1, hid), W2, b2.reshape(1, hid), wct,
      bc.reshape(1, ncls))
    return out
